# baseline (device time: 183873 ns/iter reference)
import jax
import jax.numpy as jnp
from jax import lax
from jax.experimental import pallas as pl
from jax.experimental.pallas import tpu as pltpu

N_DEV = 16
N_TOK = 2048
D_MODEL = 512
D_HID = 1024
N_EXP = 64
E_LOCAL = N_EXP // N_DEV
CHUNK = N_TOK // N_DEV
N_HOP = N_DEV - 1

RINGS = (
    {"dstep": -1, "c0": 0, "w": 128},
    {"dstep": 1, "c0": 512, "w": 128},
    {"dstep": -1, "c0": 128, "w": 128},
    {"dstep": 1, "c0": 640, "w": 128},
    {"dstep": -1, "c0": 256, "w": 128},
    {"dstep": 1, "c0": 768, "w": 128},
    {"dstep": -1, "c0": 384, "w": 128},
    {"dstep": 1, "c0": 896, "w": 128},
)


def kernel(x, router_W, route_idx, expert_W):
    def body(x_ref, rw_ref, idx_ref, ew_ref, out_ref, acc_ref, *sc):
        my = lax.axis_index("i")

        def mod(v):
            return lax.rem(v + 4 * N_DEV, N_DEV)

        def sc_of(ri):
            return sc[ri * 5 : (ri + 1) * 5]

        tgts = [mod(my - rg["dstep"]) for rg in RINGS]
        owns = [mod(my - rg["dstep"]) for rg in RINGS]
        all_descs = []
        rs_descs = [[None] * N_HOP for _ in RINGS]
        ag_descs = [[None] * N_HOP for _ in RINGS]

        def start_rs(ri, s):
            rg = RINGS[ri]
            comm1, s1s, s1r, _, _ = sc_of(ri)
            c = mod(my + rg["dstep"] * s)
            d = pltpu.make_async_remote_copy(
                src_ref=acc_ref.at[
                    pl.ds(c * CHUNK, CHUNK), pl.ds(rg["c0"], rg["w"])
                ],
                dst_ref=comm1.at[s],
                send_sem=s1s.at[s],
                recv_sem=s1r.at[s],
                device_id=(tgts[ri],),
                device_id_type=pl.DeviceIdType.MESH,
            )
            d.start()
            all_descs.append(d)
            rs_descs[ri][s] = d

        def start_ag(ri, h):
            rg = RINGS[ri]
            _, _, _, s2s, s2r = sc_of(ri)
            c_s = mod(owns[ri] + rg["dstep"] * h)
            blk = out_ref.at[pl.ds(c_s * CHUNK, CHUNK), pl.ds(rg["c0"], rg["w"])]
            d = pltpu.make_async_remote_copy(
                src_ref=blk,
                dst_ref=blk,
                send_sem=s2s.at[h],
                recv_sem=s2r.at[h],
                device_id=(tgts[ri],),
                device_id_type=pl.DeviceIdType.MESH,
            )
            d.start()
            all_descs.append(d)
            ag_descs[ri][h] = d

        step_of = [0] * len(RINGS)

        def process_step(ri):
            rg = RINGS[ri]
            t = step_of[ri]
            step_of[ri] = t + 1
            cols_sl = pl.ds(rg["c0"], rg["w"])
            if t < N_HOP:
                comm1, _, _, _, _ = sc_of(ri)
                rs_descs[ri][t].wait_recv()
                c = mod(my + rg["dstep"] * (t + 1))
                rows = pl.ds(c * CHUNK, CHUNK)
                acc_ref[rows, cols_sl] = acc_ref[rows, cols_sl] + comm1[t]
                if t + 1 < N_HOP:
                    start_rs(ri, t + 1)
                else:
                    own_rows = pl.ds(owns[ri] * CHUNK, CHUNK)
                    out_ref[own_rows, cols_sl] = acc_ref[own_rows, cols_sl]
                    start_ag(ri, 0)
            else:
                h = t - N_HOP
                ag_descs[ri][h].wait_recv()
                if h + 1 < N_HOP:
                    start_ag(ri, h + 1)

        xv = x_ref[...]
        scores = jnp.dot(xv, rw_ref[...], preferred_element_type=jnp.float32)
        m = jnp.max(scores, axis=-1, keepdims=True)
        p = jnp.exp(scores - m)
        p = p / jnp.sum(p, axis=-1, keepdims=True)
        idx = idx_ref[...]
        e0 = idx[:, 0:1]
        e1 = idx[:, 1:2]
        cols = lax.broadcasted_iota(jnp.int32, (N_TOK, N_EXP), 1)
        g0 = jnp.sum(jnp.where(cols == e0, p, 0.0), axis=-1, keepdims=True)
        g1 = jnp.sum(jnp.where(cols == e1, p, 0.0), axis=-1, keepdims=True)
        gs = g0 + g1
        xw = []
        for le in range(E_LOCAL):
            ge = my * E_LOCAL + le
            w = jnp.where(e0 == ge, g0 / gs, 0.0) + jnp.where(e1 == ge, g1 / gs, 0.0)
            xw.append(xv * w)

        for k, rg in enumerate(RINGS):
            a = jnp.zeros((N_TOK, rg["w"]), jnp.float32)
            for le in range(E_LOCAL):
                a = a + jnp.dot(
                    xw[le],
                    ew_ref[le, :, pl.ds(rg["c0"], rg["w"])],
                    preferred_element_type=jnp.float32,
                )
            acc_ref[:, pl.ds(rg["c0"], rg["w"])] = a
            start_rs(k, 0)
            for rj in range(k):
                process_step(rj)

        while any(t < 2 * N_HOP for t in step_of):
            for ri in range(len(RINGS)):
                if step_of[ri] < 2 * N_HOP:
                    process_step(ri)

        for d in all_descs:
            d.wait_send()

    scratch = [pltpu.VMEM((N_TOK, D_HID), jnp.float32)]
    for rg in RINGS:
        scratch += [
            pltpu.VMEM((N_HOP, CHUNK, rg["w"]), jnp.float32),
            pltpu.SemaphoreType.DMA((N_HOP,)),
            pltpu.SemaphoreType.DMA((N_HOP,)),
            pltpu.SemaphoreType.DMA((N_HOP,)),
            pltpu.SemaphoreType.DMA((N_HOP,)),
        ]

    return pl.pallas_call(
        body,
        out_shape=jax.ShapeDtypeStruct((N_TOK, D_HID), jnp.float32),
        in_specs=[pl.BlockSpec(memory_space=pltpu.VMEM)] * 4,
        out_specs=pl.BlockSpec(memory_space=pltpu.VMEM),
        scratch_shapes=scratch,
        compiler_params=pltpu.CompilerParams(
            vmem_limit_bytes=100 * 1024 * 1024,
        ),
    )(x, router_W, route_idx, expert_W)


# device time: 158031 ns/iter; 1.1635x vs baseline; 1.1635x over previous
import jax
import jax.numpy as jnp
from jax import lax
from jax.experimental import pallas as pl
from jax.experimental.pallas import tpu as pltpu

N_DEV = 16
N_TOK = 2048
D_MODEL = 512
D_HID = 1024
N_EXP = 64
E_LOCAL = N_EXP // N_DEV
CHUNK = N_TOK // N_DEV
N_HOP = N_DEV - 1

RINGS = (
    {"dstep": -1, "c0": 0, "w": 128},
    {"dstep": 1, "c0": 512, "w": 128},
    {"dstep": -1, "c0": 128, "w": 128},
    {"dstep": 1, "c0": 640, "w": 128},
    {"dstep": -1, "c0": 256, "w": 128},
    {"dstep": 1, "c0": 768, "w": 128},
    {"dstep": -1, "c0": 384, "w": 128},
    {"dstep": 1, "c0": 896, "w": 128},
)


def kernel(x, router_W, route_idx, expert_W):
    def body(x_ref, rw_ref, idx_ref, ew_ref, out_ref, acc_ref, *sc):
        my = lax.axis_index("i")

        def mod(v):
            return lax.rem(v + 4 * N_DEV, N_DEV)

        def sc_of(ri):
            return sc[ri * 5 : (ri + 1) * 5]

        tgts = [mod(my - rg["dstep"]) for rg in RINGS]
        owns = [mod(my - rg["dstep"]) for rg in RINGS]
        all_descs = []
        rs_descs = [[None] * N_HOP for _ in RINGS]
        ag_descs = [[None] * N_HOP for _ in RINGS]

        def start_rs(ri, s):
            rg = RINGS[ri]
            comm1, s1s, s1r, _, _ = sc_of(ri)
            c = mod(my + rg["dstep"] * s)
            d = pltpu.make_async_remote_copy(
                src_ref=acc_ref.at[
                    pl.ds(c * CHUNK, CHUNK), pl.ds(rg["c0"], rg["w"])
                ],
                dst_ref=comm1.at[s],
                send_sem=s1s.at[s],
                recv_sem=s1r.at[s],
                device_id=(tgts[ri],),
                device_id_type=pl.DeviceIdType.MESH,
            )
            d.start()
            all_descs.append(d)
            rs_descs[ri][s] = d

        def start_ag(ri, h):
            rg = RINGS[ri]
            _, _, _, s2s, s2r = sc_of(ri)
            c_s = mod(owns[ri] + rg["dstep"] * h)
            blk = out_ref.at[pl.ds(c_s * CHUNK, CHUNK), pl.ds(rg["c0"], rg["w"])]
            d = pltpu.make_async_remote_copy(
                src_ref=blk,
                dst_ref=blk,
                send_sem=s2s.at[h],
                recv_sem=s2r.at[h],
                device_id=(tgts[ri],),
                device_id_type=pl.DeviceIdType.MESH,
            )
            d.start()
            all_descs.append(d)
            ag_descs[ri][h] = d

        step_of = [0] * len(RINGS)

        def process_step(ri):
            rg = RINGS[ri]
            t = step_of[ri]
            step_of[ri] = t + 1
            cols_sl = pl.ds(rg["c0"], rg["w"])
            if t < N_HOP:
                comm1, _, _, _, _ = sc_of(ri)
                rs_descs[ri][t].wait_recv()
                c = mod(my + rg["dstep"] * (t + 1))
                rows = pl.ds(c * CHUNK, CHUNK)
                acc_ref[rows, cols_sl] = acc_ref[rows, cols_sl] + comm1[t]
                if t + 1 < N_HOP:
                    start_rs(ri, t + 1)
                else:
                    own_rows = pl.ds(owns[ri] * CHUNK, CHUNK)
                    out_ref[own_rows, cols_sl] = acc_ref[own_rows, cols_sl]
                    start_ag(ri, 0)
            else:
                h = t - N_HOP
                ag_descs[ri][h].wait_recv()
                if h + 1 < N_HOP:
                    start_ag(ri, h + 1)

        xv = x_ref[...]
        scores = jnp.dot(xv, rw_ref[...], preferred_element_type=jnp.float32)
        m = jnp.max(scores, axis=-1, keepdims=True)
        p = jnp.exp(scores - m)
        p = p / jnp.sum(p, axis=-1, keepdims=True)
        idx = idx_ref[...]
        e0 = idx[:, 0:1]
        e1 = idx[:, 1:2]
        cols = lax.broadcasted_iota(jnp.int32, (N_TOK, N_EXP), 1)
        g0 = jnp.sum(jnp.where(cols == e0, p, 0.0), axis=-1, keepdims=True)
        g1 = jnp.sum(jnp.where(cols == e1, p, 0.0), axis=-1, keepdims=True)
        gs = g0 + g1
        acc = jnp.zeros((N_TOK, D_HID), jnp.float32)
        for le in range(E_LOCAL):
            ge = my * E_LOCAL + le
            w = jnp.where(e0 == ge, g0 / gs, 0.0) + jnp.where(e1 == ge, g1 / gs, 0.0)
            acc = acc + jnp.dot(
                xv * w, ew_ref[le], preferred_element_type=jnp.float32
            )
        acc_ref[...] = acc

        for ri in range(len(RINGS)):
            start_rs(ri, 0)

        while any(t < 2 * N_HOP for t in step_of):
            for ri in range(len(RINGS)):
                if step_of[ri] < 2 * N_HOP:
                    process_step(ri)

        for d in all_descs:
            d.wait_send()

    scratch = [pltpu.VMEM((N_TOK, D_HID), jnp.float32)]
    for rg in RINGS:
        scratch += [
            pltpu.VMEM((N_HOP, CHUNK, rg["w"]), jnp.float32),
            pltpu.SemaphoreType.DMA((N_HOP,)),
            pltpu.SemaphoreType.DMA((N_HOP,)),
            pltpu.SemaphoreType.DMA((N_HOP,)),
            pltpu.SemaphoreType.DMA((N_HOP,)),
        ]

    return pl.pallas_call(
        body,
        out_shape=jax.ShapeDtypeStruct((N_TOK, D_HID), jnp.float32),
        in_specs=[pl.BlockSpec(memory_space=pltpu.VMEM)] * 4,
        out_specs=pl.BlockSpec(memory_space=pltpu.VMEM),
        scratch_shapes=scratch,
        compiler_params=pltpu.CompilerParams(
            vmem_limit_bytes=100 * 1024 * 1024,
        ),
    )(x, router_W, route_idx, expert_W)


# device time: 145813 ns/iter; 1.2610x vs baseline; 1.0838x over previous
import jax
import jax.numpy as jnp
from jax import lax
from jax.experimental import pallas as pl
from jax.experimental.pallas import tpu as pltpu

N_DEV = 16
N_TOK = 2048
D_MODEL = 512
D_HID = 1024
N_EXP = 64
E_LOCAL = N_EXP // N_DEV
CHUNK = N_TOK // N_DEV
N_HOP = N_DEV - 1

RINGS = (
    {"dstep": -1, "c0": 0, "w": 128},
    {"dstep": 1, "c0": 512, "w": 128},
    {"dstep": -1, "c0": 128, "w": 128},
    {"dstep": 1, "c0": 640, "w": 128},
    {"dstep": -1, "c0": 256, "w": 128},
    {"dstep": 1, "c0": 768, "w": 128},
    {"dstep": -1, "c0": 384, "w": 128},
    {"dstep": 1, "c0": 896, "w": 128},
)


def kernel(x, router_W, route_idx, expert_W):
    def body(x_ref, rw_ref, idx_ref, ew_ref, out_ref, acc_ref, xw_ref, *sc):
        my = lax.axis_index("i")

        def mod(v):
            return lax.rem(v + 4 * N_DEV, N_DEV)

        def sc_of(ri):
            return sc[ri * 5 : (ri + 1) * 5]

        tgts = [mod(my - rg["dstep"]) for rg in RINGS]
        owns = [mod(my - rg["dstep"]) for rg in RINGS]
        all_descs = []
        rs_descs = [[None] * N_HOP for _ in RINGS]
        ag_descs = [[None] * N_HOP for _ in RINGS]

        def start_rs(ri, s):
            rg = RINGS[ri]
            comm1, s1s, s1r, _, _ = sc_of(ri)
            c = mod(my + rg["dstep"] * s)
            d = pltpu.make_async_remote_copy(
                src_ref=acc_ref.at[
                    pl.ds(c * CHUNK, CHUNK), pl.ds(rg["c0"], rg["w"])
                ],
                dst_ref=comm1.at[s],
                send_sem=s1s.at[s],
                recv_sem=s1r.at[s],
                device_id=(tgts[ri],),
                device_id_type=pl.DeviceIdType.MESH,
            )
            d.start()
            all_descs.append(d)
            rs_descs[ri][s] = d

        def start_ag(ri, h):
            rg = RINGS[ri]
            _, _, _, s2s, s2r = sc_of(ri)
            c_s = mod(owns[ri] + rg["dstep"] * h)
            blk = out_ref.at[pl.ds(c_s * CHUNK, CHUNK), pl.ds(rg["c0"], rg["w"])]
            d = pltpu.make_async_remote_copy(
                src_ref=blk,
                dst_ref=blk,
                send_sem=s2s.at[h],
                recv_sem=s2r.at[h],
                device_id=(tgts[ri],),
                device_id_type=pl.DeviceIdType.MESH,
            )
            d.start()
            all_descs.append(d)
            ag_descs[ri][h] = d

        step_of = [0] * len(RINGS)

        def process_step(ri):
            rg = RINGS[ri]
            t = step_of[ri]
            step_of[ri] = t + 1
            cols_sl = pl.ds(rg["c0"], rg["w"])
            if t < N_HOP:
                comm1, _, _, _, _ = sc_of(ri)
                rs_descs[ri][t].wait_recv()
                c = mod(my + rg["dstep"] * (t + 1))
                rows = pl.ds(c * CHUNK, CHUNK)
                acc_ref[rows, cols_sl] = acc_ref[rows, cols_sl] + comm1[t]
                if t + 1 < N_HOP:
                    start_rs(ri, t + 1)
                else:
                    own_rows = pl.ds(owns[ri] * CHUNK, CHUNK)
                    out_ref[own_rows, cols_sl] = acc_ref[own_rows, cols_sl]
                    start_ag(ri, 0)
            else:
                h = t - N_HOP
                ag_descs[ri][h].wait_recv()
                if h + 1 < N_HOP:
                    start_ag(ri, h + 1)

        xv = x_ref[...]
        scores = jnp.dot(xv, rw_ref[...], preferred_element_type=jnp.float32)
        m = jnp.max(scores, axis=-1, keepdims=True)
        p = jnp.exp(scores - m)
        p = p / jnp.sum(p, axis=-1, keepdims=True)
        idx = idx_ref[...]
        e0 = idx[:, 0:1]
        e1 = idx[:, 1:2]
        cols = lax.broadcasted_iota(jnp.int32, (N_TOK, N_EXP), 1)
        g0 = jnp.sum(jnp.where(cols == e0, p, 0.0), axis=-1, keepdims=True)
        g1 = jnp.sum(jnp.where(cols == e1, p, 0.0), axis=-1, keepdims=True)
        gs = g0 + g1
        xws = []
        for le in range(E_LOCAL):
            ge = my * E_LOCAL + le
            w = jnp.where(e0 == ge, g0 / gs, 0.0) + jnp.where(e1 == ge, g1 / gs, 0.0)
            xws.append(xv * w)
        xw_ref[...] = jnp.concatenate(xws, axis=1)
        ew_flat = ew_ref[...].reshape(E_LOCAL * D_MODEL, D_HID)

        def compute_chunk(c):
            rows = pl.ds(c * CHUNK, CHUNK)
            acc_ref[rows, :] = jnp.dot(
                xw_ref[rows, :], ew_flat, preferred_element_type=jnp.float32
            )

        compute_chunk(my)
        for ri in range(len(RINGS)):
            start_rs(ri, 0)
        for s in range(N_HOP):
            off = s + 1
            if off <= N_DEV // 2:
                compute_chunk(mod(my - off))
                if off < N_DEV // 2:
                    compute_chunk(mod(my + off))
            for ri in range(len(RINGS)):
                process_step(ri)

        while any(t < 2 * N_HOP for t in step_of):
            for ri in range(len(RINGS)):
                if step_of[ri] < 2 * N_HOP:
                    process_step(ri)

        for d in all_descs:
            d.wait_send()

    scratch = [
        pltpu.VMEM((N_TOK, D_HID), jnp.float32),
        pltpu.VMEM((N_TOK, E_LOCAL * D_MODEL), jnp.float32),
    ]
    for rg in RINGS:
        scratch += [
            pltpu.VMEM((N_HOP, CHUNK, rg["w"]), jnp.float32),
            pltpu.SemaphoreType.DMA((N_HOP,)),
            pltpu.SemaphoreType.DMA((N_HOP,)),
            pltpu.SemaphoreType.DMA((N_HOP,)),
            pltpu.SemaphoreType.DMA((N_HOP,)),
        ]

    return pl.pallas_call(
        body,
        out_shape=jax.ShapeDtypeStruct((N_TOK, D_HID), jnp.float32),
        in_specs=[pl.BlockSpec(memory_space=pltpu.VMEM)] * 4,
        out_specs=pl.BlockSpec(memory_space=pltpu.VMEM),
        scratch_shapes=scratch,
        compiler_params=pltpu.CompilerParams(
            vmem_limit_bytes=100 * 1024 * 1024,
        ),
    )(x, router_W, route_idx, expert_W)
